# Initial kernel scaffold; baseline (speedup 1.0000x reference)
import jax
import jax.numpy as jnp
from jax.experimental import pallas as pl


def kernel(x, edge_index, batch, W_gcn, b_gcn, W_enc, b_enc, W_dec, b_dec):
    raise NotImplementedError("write your pallas kernel here")


# speedup vs baseline: 16.0386x; 16.0386x over previous
import functools

import jax
import jax.numpy as jnp
from jax import lax
from jax.experimental import pallas as pl
from jax.experimental.pallas import tpu as pltpu
from jax.experimental.pallas import tpu_sc as plsc

N = 10000
E = 320000
IN_CH = 128
HID = 128
LAT = 64
G = 64

NC = 2
NS = 16
NW = NC * NS
EPW = E // NW
CH = 80
NCHUNK = EPW // CH
NPAD = 10240
RPT = NPAD // NS

_sc_mesh = plsc.VectorSubcoreMesh(
    core_axis_name="c", subcore_axis_name="s", num_cores=NC, num_subcores=NS
)


@functools.partial(
    pl.kernel,
    out_type=jax.ShapeDtypeStruct((NC, NPAD, HID), jnp.float32),
    mesh=_sc_mesh,
    scratch_types=[
        pltpu.VMEM((NCHUNK, CH), jnp.int32),
        pltpu.VMEM((CH, HID), jnp.float32),
        pltpu.VMEM_SHARED((NPAD, HID), jnp.float32),
    ],
)
def _sc_degree(dst_hbm, ones_hbm, zeros_hbm, out_hbm, idx_v, ones_v, hist_s):
    c = lax.axis_index("c")
    s = lax.axis_index("s")
    wid = s * NC + c
    pltpu.sync_copy(dst_hbm.at[wid], idx_v)
    pltpu.sync_copy(ones_hbm, ones_v)
    pltpu.sync_copy(zeros_hbm.at[pl.ds(s * RPT, RPT)], hist_s.at[pl.ds(s * RPT, RPT)])
    plsc.subcore_barrier()

    def body(ci, carry):
        pltpu.sync_copy(ones_v, hist_s.at[idx_v.at[ci]], add=True)
        return carry

    lax.fori_loop(0, NCHUNK, body, 0)
    plsc.subcore_barrier()
    pltpu.sync_copy(
        hist_s.at[pl.ds(s * RPT, RPT)], out_hbm.at[c, pl.ds(s * RPT, RPT)]
    )


@functools.partial(
    pl.kernel,
    out_type=jax.ShapeDtypeStruct((NC, NPAD, HID), jnp.float32),
    mesh=_sc_mesh,
    scratch_types=[
        pltpu.VMEM((NCHUNK, CH), jnp.int32),
        pltpu.VMEM((NCHUNK, CH), jnp.int32),
        pltpu.VMEM((CH, HID), jnp.float32),
        pltpu.VMEM_SHARED((NPAD, HID), jnp.float32),
        pltpu.SemaphoreType.DMA,
    ],
)
def _sc_aggregate(
    src_hbm, dst_hbm, y_hbm, zeros_hbm, out_hbm, src_v, dst_v, rows_v, agg_s, sem
):
    c = lax.axis_index("c")
    s = lax.axis_index("s")
    wid = s * NC + c
    pltpu.sync_copy(src_hbm.at[wid], src_v)
    pltpu.sync_copy(dst_hbm.at[wid], dst_v)
    pltpu.sync_copy(zeros_hbm.at[pl.ds(s * RPT, RPT)], agg_s.at[pl.ds(s * RPT, RPT)])
    plsc.subcore_barrier()

    def body(ci, carry):
        pltpu.async_copy(y_hbm.at[src_v.at[ci]], rows_v, sem).wait()
        pltpu.sync_copy(rows_v, agg_s.at[dst_v.at[ci]], add=True)
        return carry

    lax.fori_loop(0, NCHUNK, body, 0)
    plsc.subcore_barrier()
    pltpu.sync_copy(
        agg_s.at[pl.ds(s * RPT, RPT)], out_hbm.at[c, pl.ds(s * RPT, RPT)]
    )


RB = 1000
NBL = N // RB


def _tc_pre_body(x_ref, w_ref, h0_ref, h1_ref, xw_ref, y_ref):
    deg = h0_ref[0][:, :1] + h1_ref[0][:, :1] + 1.0
    xw = jnp.dot(x_ref[...], w_ref[...], preferred_element_type=jnp.float32)
    xw_ref[...] = xw
    y_ref[...] = xw * lax.rsqrt(deg)


def _tc_pre(x, W, hist):
    return pl.pallas_call(
        _tc_pre_body,
        grid=(NBL,),
        in_specs=[
            pl.BlockSpec((RB, IN_CH), lambda i: (i, 0)),
            pl.BlockSpec((IN_CH, HID), lambda i: (0, 0)),
            pl.BlockSpec((1, RB, HID), lambda i: (0, i, 0)),
            pl.BlockSpec((1, RB, HID), lambda i: (1, i, 0)),
        ],
        out_specs=[
            pl.BlockSpec((RB, HID), lambda i: (i, 0)),
            pl.BlockSpec((RB, HID), lambda i: (i, 0)),
        ],
        out_shape=[
            jax.ShapeDtypeStruct((N, HID), jnp.float32),
            jax.ShapeDtypeStruct((N, HID), jnp.float32),
        ],
    )(x, W, hist, hist)


def _tc_enc_body(a0, a1, xw_ref, h0, h1, bg, we, be, b_ref, z_ref, zg_ref, cnt_ref):
    i = pl.program_id(0)
    deg = h0[0][:, :1] + h1[0][:, :1] + 1.0
    dis = lax.rsqrt(deg)
    out = dis * (a0[0] + a1[0]) + xw_ref[...] * (1.0 / deg) + bg[...]
    hrelu = jnp.maximum(out, 0.0)
    z = jnp.dot(hrelu, we[...], preferred_element_type=jnp.float32) + be[...]
    z_ref[...] = z
    b2 = b_ref[0]
    S = (lax.broadcasted_iota(jnp.int32, (G, RB), 0) == b2).astype(jnp.float32)
    zg = jnp.dot(S, z, preferred_element_type=jnp.float32)
    csum = jnp.sum(S, axis=1, keepdims=True)

    @pl.when(i == 0)
    def _():
        zg_ref[...] = jnp.zeros_like(zg_ref)
        cnt_ref[...] = jnp.zeros_like(cnt_ref)

    zg_ref[...] += zg
    cnt_ref[...] += jnp.broadcast_to(csum, (G, G))


def _tc_encode(agg, xw, hist, bg, We, be, batch3):
    return pl.pallas_call(
        _tc_enc_body,
        grid=(NBL,),
        in_specs=[
            pl.BlockSpec((1, RB, HID), lambda i: (0, i, 0)),
            pl.BlockSpec((1, RB, HID), lambda i: (1, i, 0)),
            pl.BlockSpec((RB, HID), lambda i: (i, 0)),
            pl.BlockSpec((1, RB, HID), lambda i: (0, i, 0)),
            pl.BlockSpec((1, RB, HID), lambda i: (1, i, 0)),
            pl.BlockSpec((1, HID), lambda i: (0, 0)),
            pl.BlockSpec((HID, LAT), lambda i: (0, 0)),
            pl.BlockSpec((1, LAT), lambda i: (0, 0)),
            pl.BlockSpec((1, 1, RB), lambda i: (i, 0, 0)),
        ],
        out_specs=[
            pl.BlockSpec((RB, LAT), lambda i: (i, 0)),
            pl.BlockSpec((G, LAT), lambda i: (0, 0)),
            pl.BlockSpec((G, G), lambda i: (0, 0)),
        ],
        out_shape=[
            jax.ShapeDtypeStruct((N, LAT), jnp.float32),
            jax.ShapeDtypeStruct((G, LAT), jnp.float32),
            jax.ShapeDtypeStruct((G, G), jnp.float32),
        ],
    )(agg, agg, xw, hist, hist, bg, We, be, batch3)


RC = 400
NBC = N // RC


def _tc_dec_body(zb, zfull, zgs, cnt, b_ref, wd, bd, a_ref, x_ref, zgr_ref):
    zgraph = zgs[...] / jnp.maximum(cnt[...], 1.0)
    zgr_ref[...] = zgraph
    b2 = b_ref[0]
    St = (lax.broadcasted_iota(jnp.int32, (G, RC), 0) == b2).astype(jnp.float32)
    zrep = lax.dot_general(
        St, zgraph, (((0,), (0,)), ((), ())), preferred_element_type=jnp.float32
    )
    x_ref[...] = (
        jnp.dot(zrep, wd[...], preferred_element_type=jnp.float32) + bd[...]
    )
    logits = lax.dot_general(
        zb[...], zfull[...], (((1,), (1,)), ((), ())),
        preferred_element_type=jnp.float32,
    )
    a_ref[...] = 1.0 / (1.0 + jnp.exp(-logits))


def _tc_decode(z_node, zgs, cnt, batch3, Wd, bd):
    return pl.pallas_call(
        _tc_dec_body,
        grid=(NBC,),
        in_specs=[
            pl.BlockSpec((RC, LAT), lambda i: (i, 0)),
            pl.BlockSpec((N, LAT), lambda i: (0, 0)),
            pl.BlockSpec((G, LAT), lambda i: (0, 0)),
            pl.BlockSpec((G, G), lambda i: (0, 0)),
            pl.BlockSpec((1, 1, RC), lambda i: (i, 0, 0)),
            pl.BlockSpec((LAT, IN_CH), lambda i: (0, 0)),
            pl.BlockSpec((1, IN_CH), lambda i: (0, 0)),
        ],
        out_specs=[
            pl.BlockSpec((RC, N), lambda i: (i, 0)),
            pl.BlockSpec((RC, IN_CH), lambda i: (i, 0)),
            pl.BlockSpec((G, LAT), lambda i: (0, 0)),
        ],
        out_shape=[
            jax.ShapeDtypeStruct((N, N), jnp.float32),
            jax.ShapeDtypeStruct((N, IN_CH), jnp.float32),
            jax.ShapeDtypeStruct((G, LAT), jnp.float32),
        ],
    )(z_node, z_node, zgs, cnt, batch3, Wd, bd)


def kernel(x, edge_index, batch, W_gcn, b_gcn, W_enc, b_enc, W_dec, b_dec):
    src2 = edge_index[0].reshape(NW, NCHUNK, CH)
    dst2 = edge_index[1].reshape(NW, NCHUNK, CH)
    ones128 = jnp.ones((CH, HID), jnp.float32)
    zeros128 = jnp.zeros((NPAD, HID), jnp.float32)

    hist = _sc_degree(dst2, ones128, zeros128)
    xw, y = _tc_pre(x, W_gcn, hist)
    agg = _sc_aggregate(src2, dst2, y, zeros128)
    z_node, zg_sum, cnt = _tc_encode(
        agg, xw, hist,
        b_gcn.reshape(1, HID), W_enc, b_enc.reshape(1, LAT),
        batch.reshape(NBL, 1, RB),
    )
    a_hat, x_hat, z_graph = _tc_decode(
        z_node, zg_sum, cnt, batch.reshape(NBC, 1, RC),
        W_dec, b_dec.reshape(1, IN_CH),
    )
    return z_node, z_graph, x_hat, a_hat


# speedup vs baseline: 17.1113x; 1.0669x over previous
import functools

import jax
import jax.numpy as jnp
from jax import lax
from jax.experimental import pallas as pl
from jax.experimental.pallas import tpu as pltpu
from jax.experimental.pallas import tpu_sc as plsc

N = 10000
E = 320000
IN_CH = 128
HID = 128
LAT = 64
G = 64

NC = 2
NS = 16
NW = NC * NS
EPW = E // NW
CH = 40
NCHUNK = EPW // CH
NPAD = 10240
RPT = NPAD // NS

_sc_mesh = plsc.VectorSubcoreMesh(
    core_axis_name="c", subcore_axis_name="s", num_cores=NC, num_subcores=NS
)


@functools.partial(
    pl.kernel,
    out_type=jax.ShapeDtypeStruct((NC, NPAD, HID), jnp.float32),
    mesh=_sc_mesh,
    scratch_types=[
        pltpu.VMEM((NCHUNK // 2, CH), jnp.int32),
        pltpu.VMEM((CH, HID), jnp.float32),
        pltpu.VMEM_SHARED((NPAD, HID), jnp.float32),
    ],
)
def _sc_degree(dst_hbm, ones_hbm, zeros_hbm, out_hbm, idx_v, ones_v, hist_s):
    c = lax.axis_index("c")
    s = lax.axis_index("s")
    wid = s * NC + c
    pltpu.sync_copy(ones_hbm, ones_v)
    pltpu.sync_copy(zeros_hbm.at[pl.ds(s * RPT, RPT)], hist_s.at[pl.ds(s * RPT, RPT)])
    plsc.subcore_barrier()

    def body(ci, carry):
        pltpu.sync_copy(ones_v, hist_s.at[idx_v.at[ci]], add=True)
        return carry

    for h in range(2):
        pltpu.sync_copy(dst_hbm.at[wid, h], idx_v)
        lax.fori_loop(0, NCHUNK // 2, body, 0)
    plsc.subcore_barrier()
    pltpu.sync_copy(
        hist_s.at[pl.ds(s * RPT, RPT)], out_hbm.at[c, pl.ds(s * RPT, RPT)]
    )


@functools.partial(
    pl.kernel,
    out_type=jax.ShapeDtypeStruct((NC, NPAD, HID), jnp.float32),
    mesh=_sc_mesh,
    scratch_types=[
        pltpu.VMEM((NCHUNK // 2, CH), jnp.int32),
        pltpu.VMEM((NCHUNK // 2, CH), jnp.int32),
        pltpu.VMEM((CH, HID), jnp.float32),
        pltpu.VMEM((CH, HID), jnp.float32),
        pltpu.VMEM_SHARED((NPAD, HID), jnp.float32),
        pltpu.SemaphoreType.DMA,
        pltpu.SemaphoreType.DMA,
    ],
)
def _sc_aggregate(
    src_hbm, dst_hbm, y_hbm, zeros_hbm, out_hbm,
    src_v, dst_v, rows_a, rows_b, agg_s, sem_a, sem_b,
):
    c = lax.axis_index("c")
    s = lax.axis_index("s")
    wid = s * NC + c
    NH = NCHUNK // 2
    pltpu.sync_copy(zeros_hbm.at[pl.ds(s * RPT, RPT)], agg_s.at[pl.ds(s * RPT, RPT)])
    plsc.subcore_barrier()

    for h in range(2):
        pltpu.sync_copy(src_hbm.at[wid, h], src_v)
        pltpu.sync_copy(dst_hbm.at[wid, h], dst_v)

        pltpu.async_copy(y_hbm.at[src_v.at[0]], rows_a, sem_a)

        def pair(j, carry):
            c0 = 2 * j
            c1 = c0 + 1
            pltpu.async_copy(y_hbm.at[src_v.at[c1]], rows_b, sem_b)
            pltpu.make_async_copy(y_hbm.at[src_v.at[c0]], rows_a, sem_a).wait()
            pltpu.sync_copy(rows_a, agg_s.at[dst_v.at[c0]], add=True)

            @pl.when(c0 + 2 < NH)
            def _():
                pltpu.async_copy(y_hbm.at[src_v.at[c0 + 2]], rows_a, sem_a)

            pltpu.make_async_copy(y_hbm.at[src_v.at[c1]], rows_b, sem_b).wait()
            pltpu.sync_copy(rows_b, agg_s.at[dst_v.at[c1]], add=True)
            return carry

        lax.fori_loop(0, NH // 2, pair, 0)
        pltpu.make_async_copy(y_hbm.at[src_v.at[NH - 1]], rows_a, sem_a).wait()
        pltpu.sync_copy(rows_a, agg_s.at[dst_v.at[NH - 1]], add=True)
    plsc.subcore_barrier()
    pltpu.sync_copy(
        agg_s.at[pl.ds(s * RPT, RPT)], out_hbm.at[c, pl.ds(s * RPT, RPT)]
    )


RB = 1000
NBL = N // RB


def _tc_pre_body(x_ref, w_ref, h0_ref, h1_ref, xw_ref, y_ref):
    deg = h0_ref[0][:, :1] + h1_ref[0][:, :1] + 1.0
    xw = jnp.dot(x_ref[...], w_ref[...], preferred_element_type=jnp.float32)
    xw_ref[...] = xw
    y_ref[...] = xw * lax.rsqrt(deg)


def _tc_pre(x, W, hist):
    return pl.pallas_call(
        _tc_pre_body,
        grid=(NBL,),
        in_specs=[
            pl.BlockSpec((RB, IN_CH), lambda i: (i, 0)),
            pl.BlockSpec((IN_CH, HID), lambda i: (0, 0)),
            pl.BlockSpec((1, RB, HID), lambda i: (0, i, 0)),
            pl.BlockSpec((1, RB, HID), lambda i: (1, i, 0)),
        ],
        out_specs=[
            pl.BlockSpec((RB, HID), lambda i: (i, 0)),
            pl.BlockSpec((RB, HID), lambda i: (i, 0)),
        ],
        out_shape=[
            jax.ShapeDtypeStruct((N, HID), jnp.float32),
            jax.ShapeDtypeStruct((N, HID), jnp.float32),
        ],
    )(x, W, hist, hist)


def _tc_enc_body(a0, a1, xw_ref, h0, h1, bg, we, be, b_ref, z_ref, zg_ref, cnt_ref):
    i = pl.program_id(0)
    deg = h0[0][:, :1] + h1[0][:, :1] + 1.0
    dis = lax.rsqrt(deg)
    out = dis * (a0[0] + a1[0]) + xw_ref[...] * (1.0 / deg) + bg[...]
    hrelu = jnp.maximum(out, 0.0)
    z = jnp.dot(hrelu, we[...], preferred_element_type=jnp.float32) + be[...]
    z_ref[...] = z
    b2 = b_ref[0]
    S = (lax.broadcasted_iota(jnp.int32, (G, RB), 0) == b2).astype(jnp.float32)
    zg = jnp.dot(S, z, preferred_element_type=jnp.float32)
    csum = jnp.sum(S, axis=1, keepdims=True)

    @pl.when(i == 0)
    def _():
        zg_ref[...] = jnp.zeros_like(zg_ref)
        cnt_ref[...] = jnp.zeros_like(cnt_ref)

    zg_ref[...] += zg
    cnt_ref[...] += jnp.broadcast_to(csum, (G, G))


def _tc_encode(agg, xw, hist, bg, We, be, batch3):
    return pl.pallas_call(
        _tc_enc_body,
        grid=(NBL,),
        in_specs=[
            pl.BlockSpec((1, RB, HID), lambda i: (0, i, 0)),
            pl.BlockSpec((1, RB, HID), lambda i: (1, i, 0)),
            pl.BlockSpec((RB, HID), lambda i: (i, 0)),
            pl.BlockSpec((1, RB, HID), lambda i: (0, i, 0)),
            pl.BlockSpec((1, RB, HID), lambda i: (1, i, 0)),
            pl.BlockSpec((1, HID), lambda i: (0, 0)),
            pl.BlockSpec((HID, LAT), lambda i: (0, 0)),
            pl.BlockSpec((1, LAT), lambda i: (0, 0)),
            pl.BlockSpec((1, 1, RB), lambda i: (i, 0, 0)),
        ],
        out_specs=[
            pl.BlockSpec((RB, LAT), lambda i: (i, 0)),
            pl.BlockSpec((G, LAT), lambda i: (0, 0)),
            pl.BlockSpec((G, G), lambda i: (0, 0)),
        ],
        out_shape=[
            jax.ShapeDtypeStruct((N, LAT), jnp.float32),
            jax.ShapeDtypeStruct((G, LAT), jnp.float32),
            jax.ShapeDtypeStruct((G, G), jnp.float32),
        ],
    )(agg, agg, xw, hist, hist, bg, We, be, batch3)


RC = 400
NBC = N // RC


def _tc_dec_body(zb, zfull, zgs, cnt, b_ref, wd, bd, a_ref, x_ref, zgr_ref):
    zgraph = zgs[...] / jnp.maximum(cnt[...], 1.0)
    zgr_ref[...] = zgraph
    b2 = b_ref[0]
    St = (lax.broadcasted_iota(jnp.int32, (G, RC), 0) == b2).astype(jnp.float32)
    zrep = lax.dot_general(
        St, zgraph, (((0,), (0,)), ((), ())), preferred_element_type=jnp.float32
    )
    x_ref[...] = (
        jnp.dot(zrep, wd[...], preferred_element_type=jnp.float32) + bd[...]
    )
    logits = lax.dot_general(
        zb[...], zfull[...], (((1,), (1,)), ((), ())),
        preferred_element_type=jnp.float32,
    )
    a_ref[...] = 1.0 / (1.0 + jnp.exp(-logits))


def _tc_decode(z_node, zgs, cnt, batch3, Wd, bd):
    return pl.pallas_call(
        _tc_dec_body,
        grid=(NBC,),
        in_specs=[
            pl.BlockSpec((RC, LAT), lambda i: (i, 0)),
            pl.BlockSpec((N, LAT), lambda i: (0, 0)),
            pl.BlockSpec((G, LAT), lambda i: (0, 0)),
            pl.BlockSpec((G, G), lambda i: (0, 0)),
            pl.BlockSpec((1, 1, RC), lambda i: (i, 0, 0)),
            pl.BlockSpec((LAT, IN_CH), lambda i: (0, 0)),
            pl.BlockSpec((1, IN_CH), lambda i: (0, 0)),
        ],
        out_specs=[
            pl.BlockSpec((RC, N), lambda i: (i, 0)),
            pl.BlockSpec((RC, IN_CH), lambda i: (i, 0)),
            pl.BlockSpec((G, LAT), lambda i: (0, 0)),
        ],
        out_shape=[
            jax.ShapeDtypeStruct((N, N), jnp.float32),
            jax.ShapeDtypeStruct((N, IN_CH), jnp.float32),
            jax.ShapeDtypeStruct((G, LAT), jnp.float32),
        ],
    )(z_node, z_node, zgs, cnt, batch3, Wd, bd)


def kernel(x, edge_index, batch, W_gcn, b_gcn, W_enc, b_enc, W_dec, b_dec):
    src2 = edge_index[0].reshape(NW, 2, NCHUNK // 2, CH)
    dst2 = edge_index[1].reshape(NW, 2, NCHUNK // 2, CH)
    ones128 = jnp.ones((CH, HID), jnp.float32)
    zeros128 = jnp.zeros((NPAD, HID), jnp.float32)

    hist = _sc_degree(dst2, ones128, zeros128)
    xw, y = _tc_pre(x, W_gcn, hist)
    agg = _sc_aggregate(src2, dst2, y, zeros128)
    z_node, zg_sum, cnt = _tc_encode(
        agg, xw, hist,
        b_gcn.reshape(1, HID), W_enc, b_enc.reshape(1, LAT),
        batch.reshape(NBL, 1, RB),
    )
    a_hat, x_hat, z_graph = _tc_decode(
        z_node, zg_sum, cnt, batch.reshape(NBC, 1, RC),
        W_dec, b_dec.reshape(1, IN_CH),
    )
    return z_node, z_graph, x_hat, a_hat


# speedup vs baseline: 18.5463x; 1.0839x over previous
import functools

import jax
import jax.numpy as jnp
from jax import lax
from jax.experimental import pallas as pl
from jax.experimental.pallas import tpu as pltpu
from jax.experimental.pallas import tpu_sc as plsc

N = 10000
E = 320000
IN_CH = 128
HID = 128
LAT = 64
G = 64

NC = 2
NS = 16
NW = NC * NS
EPW = E // NW
CH = 40
NCHUNK = EPW // CH
NPAD = 10240
RPT = NPAD // NS

_sc_mesh = plsc.VectorSubcoreMesh(
    core_axis_name="c", subcore_axis_name="s", num_cores=NC, num_subcores=NS
)


@functools.partial(
    pl.kernel,
    out_type=jax.ShapeDtypeStruct((NC, NPAD, HID), jnp.float32),
    mesh=_sc_mesh,
    scratch_types=[
        pltpu.VMEM((NCHUNK // 2, CH), jnp.int32),
        pltpu.VMEM((CH, HID), jnp.float32),
        pltpu.VMEM_SHARED((NPAD, HID), jnp.float32),
    ],
)
def _sc_degree(dst_hbm, ones_hbm, zeros_hbm, out_hbm, idx_v, ones_v, hist_s):
    c = lax.axis_index("c")
    s = lax.axis_index("s")
    wid = s * NC + c
    pltpu.sync_copy(ones_hbm, ones_v)
    pltpu.sync_copy(zeros_hbm.at[pl.ds(s * RPT, RPT)], hist_s.at[pl.ds(s * RPT, RPT)])
    plsc.subcore_barrier()

    def body(ci, carry):
        pltpu.sync_copy(ones_v, hist_s.at[idx_v.at[ci]], add=True)
        return carry

    for h in range(2):
        pltpu.sync_copy(dst_hbm.at[wid, h], idx_v)
        lax.fori_loop(0, NCHUNK // 2, body, 0)
    plsc.subcore_barrier()
    pltpu.sync_copy(
        hist_s.at[pl.ds(s * RPT, RPT)], out_hbm.at[c, pl.ds(s * RPT, RPT)]
    )


@functools.partial(
    pl.kernel,
    out_type=jax.ShapeDtypeStruct((NC, N, HID), jnp.float32),
    mesh=_sc_mesh,
    scratch_types=[
        pltpu.VMEM((2, NCHUNK // 5, CH), jnp.int32),
        pltpu.VMEM((5, CH, HID), jnp.float32),
        [pltpu.SemaphoreType.DMA] * 5,
        [pltpu.SemaphoreType.DMA] * 5,
        pltpu.VMEM_SHARED((N, HID), jnp.float32),
    ],
)
def _sc_aggregate(
    src_hbm, dst_hbm, y_hbm, zeros_hbm, out_hbm,
    sd_v, rows5, gsem, ssem, agg_s,
):
    c = lax.axis_index("c")
    s = lax.axis_index("s")
    wid = s * NC + c
    NH = NCHUNK // 5
    NB = 5
    R0 = 624

    @pl.when(s < NS - 1)
    def _():
        pltpu.sync_copy(zeros_hbm.at[pl.ds(s * R0, R0)], agg_s.at[pl.ds(s * R0, R0)])

    @pl.when(s == NS - 1)
    def _():
        pltpu.sync_copy(
            zeros_hbm.at[pl.ds((NS - 1) * R0, N - (NS - 1) * R0)],
            agg_s.at[pl.ds((NS - 1) * R0, N - (NS - 1) * R0)],
        )

    plsc.subcore_barrier()

    for h in range(5):
        pltpu.sync_copy(src_hbm.at[wid, h], sd_v.at[0])
        pltpu.sync_copy(dst_hbm.at[wid, h], sd_v.at[1])

        for b in range(NB):
            pltpu.async_copy(y_hbm.at[sd_v.at[0, b]], rows5.at[b], gsem[b])

        def group(g, carry):
            base = g * NB
            for b in range(NB):
                ci = base + b
                pltpu.make_async_copy(
                    y_hbm.at[sd_v.at[0, ci]], rows5.at[b], gsem[b]
                ).wait()
                pltpu.async_copy(
                    rows5.at[b], agg_s.at[sd_v.at[1, ci]], ssem[b], add=True
                )
            for b in range(NB):
                ci = base + b
                pltpu.make_async_copy(
                    rows5.at[b], agg_s.at[sd_v.at[1, ci]], ssem[b]
                ).wait()

                @pl.when(base + NB + b < NH)
                def _():
                    pltpu.async_copy(
                        y_hbm.at[sd_v.at[0, base + NB + b]], rows5.at[b], gsem[b]
                    )
            return carry

        lax.fori_loop(0, NH // NB, group, 0)
    plsc.subcore_barrier()

    @pl.when(s < NS - 1)
    def _():
        pltpu.sync_copy(
            agg_s.at[pl.ds(s * R0, R0)], out_hbm.at[c, pl.ds(s * R0, R0)]
        )

    @pl.when(s == NS - 1)
    def _():
        pltpu.sync_copy(
            agg_s.at[pl.ds((NS - 1) * R0, N - (NS - 1) * R0)],
            out_hbm.at[c, pl.ds((NS - 1) * R0, N - (NS - 1) * R0)],
        )


RB = 1000
NBL = N // RB


def _tc_pre_body(x_ref, w_ref, h0_ref, h1_ref, xw_ref, y_ref):
    deg = h0_ref[0][:, :1] + h1_ref[0][:, :1] + 1.0
    xw = jnp.dot(x_ref[...], w_ref[...], preferred_element_type=jnp.float32)
    xw_ref[...] = xw
    y_ref[...] = xw * lax.rsqrt(deg)


def _tc_pre(x, W, hist):
    return pl.pallas_call(
        _tc_pre_body,
        grid=(NBL,),
        in_specs=[
            pl.BlockSpec((RB, IN_CH), lambda i: (i, 0)),
            pl.BlockSpec((IN_CH, HID), lambda i: (0, 0)),
            pl.BlockSpec((1, RB, HID), lambda i: (0, i, 0)),
            pl.BlockSpec((1, RB, HID), lambda i: (1, i, 0)),
        ],
        out_specs=[
            pl.BlockSpec((RB, HID), lambda i: (i, 0)),
            pl.BlockSpec((RB, HID), lambda i: (i, 0)),
        ],
        out_shape=[
            jax.ShapeDtypeStruct((N, HID), jnp.float32),
            jax.ShapeDtypeStruct((N, HID), jnp.float32),
        ],
    )(x, W, hist, hist)


def _tc_enc_body(a0, a1, xw_ref, h0, h1, bg, we, be, b_ref, z_ref, zg_ref, cnt_ref):
    i = pl.program_id(0)
    deg = h0[0][:, :1] + h1[0][:, :1] + 1.0
    dis = lax.rsqrt(deg)
    out = dis * (a0[0] + a1[0]) + xw_ref[...] * (1.0 / deg) + bg[...]
    hrelu = jnp.maximum(out, 0.0)
    z = jnp.dot(hrelu, we[...], preferred_element_type=jnp.float32) + be[...]
    z_ref[...] = z
    b2 = b_ref[0]
    S = (lax.broadcasted_iota(jnp.int32, (G, RB), 0) == b2).astype(jnp.float32)
    zg = jnp.dot(S, z, preferred_element_type=jnp.float32)
    csum = jnp.sum(S, axis=1, keepdims=True)

    @pl.when(i == 0)
    def _():
        zg_ref[...] = jnp.zeros_like(zg_ref)
        cnt_ref[...] = jnp.zeros_like(cnt_ref)

    zg_ref[...] += zg
    cnt_ref[...] += jnp.broadcast_to(csum, (G, G))


def _tc_encode(agg, xw, hist, bg, We, be, batch3):
    return pl.pallas_call(
        _tc_enc_body,
        grid=(NBL,),
        in_specs=[
            pl.BlockSpec((1, RB, HID), lambda i: (0, i, 0)),
            pl.BlockSpec((1, RB, HID), lambda i: (1, i, 0)),
            pl.BlockSpec((RB, HID), lambda i: (i, 0)),
            pl.BlockSpec((1, RB, HID), lambda i: (0, i, 0)),
            pl.BlockSpec((1, RB, HID), lambda i: (1, i, 0)),
            pl.BlockSpec((1, HID), lambda i: (0, 0)),
            pl.BlockSpec((HID, LAT), lambda i: (0, 0)),
            pl.BlockSpec((1, LAT), lambda i: (0, 0)),
            pl.BlockSpec((1, 1, RB), lambda i: (i, 0, 0)),
        ],
        out_specs=[
            pl.BlockSpec((RB, LAT), lambda i: (i, 0)),
            pl.BlockSpec((G, LAT), lambda i: (0, 0)),
            pl.BlockSpec((G, G), lambda i: (0, 0)),
        ],
        out_shape=[
            jax.ShapeDtypeStruct((N, LAT), jnp.float32),
            jax.ShapeDtypeStruct((G, LAT), jnp.float32),
            jax.ShapeDtypeStruct((G, G), jnp.float32),
        ],
    )(agg, agg, xw, hist, hist, bg, We, be, batch3)


RC = 400
NBC = N // RC


def _tc_dec_body(zb, zfull, zgs, cnt, b_ref, wd, bd, a_ref, x_ref, zgr_ref):
    zgraph = zgs[...] / jnp.maximum(cnt[...], 1.0)
    zgr_ref[...] = zgraph
    b2 = b_ref[0]
    St = (lax.broadcasted_iota(jnp.int32, (G, RC), 0) == b2).astype(jnp.float32)
    zrep = lax.dot_general(
        St, zgraph, (((0,), (0,)), ((), ())), preferred_element_type=jnp.float32
    )
    x_ref[...] = (
        jnp.dot(zrep, wd[...], preferred_element_type=jnp.float32) + bd[...]
    )
    logits = lax.dot_general(
        zb[...], zfull[...], (((1,), (1,)), ((), ())),
        preferred_element_type=jnp.float32,
    )
    a_ref[...] = 1.0 / (1.0 + jnp.exp(-logits))


def _tc_decode(z_node, zgs, cnt, batch3, Wd, bd):
    return pl.pallas_call(
        _tc_dec_body,
        grid=(NBC,),
        in_specs=[
            pl.BlockSpec((RC, LAT), lambda i: (i, 0)),
            pl.BlockSpec((N, LAT), lambda i: (0, 0)),
            pl.BlockSpec((G, LAT), lambda i: (0, 0)),
            pl.BlockSpec((G, G), lambda i: (0, 0)),
            pl.BlockSpec((1, 1, RC), lambda i: (i, 0, 0)),
            pl.BlockSpec((LAT, IN_CH), lambda i: (0, 0)),
            pl.BlockSpec((1, IN_CH), lambda i: (0, 0)),
        ],
        out_specs=[
            pl.BlockSpec((RC, N), lambda i: (i, 0)),
            pl.BlockSpec((RC, IN_CH), lambda i: (i, 0)),
            pl.BlockSpec((G, LAT), lambda i: (0, 0)),
        ],
        out_shape=[
            jax.ShapeDtypeStruct((N, N), jnp.float32),
            jax.ShapeDtypeStruct((N, IN_CH), jnp.float32),
            jax.ShapeDtypeStruct((G, LAT), jnp.float32),
        ],
    )(z_node, z_node, zgs, cnt, batch3, Wd, bd)


def kernel(x, edge_index, batch, W_gcn, b_gcn, W_enc, b_enc, W_dec, b_dec):
    src2 = edge_index[0].reshape(NW, 5, NCHUNK // 5, CH)
    dst2 = edge_index[1].reshape(NW, 5, NCHUNK // 5, CH)
    dst2d = edge_index[1].reshape(NW, 2, NCHUNK // 2, CH)
    ones128 = jnp.ones((CH, HID), jnp.float32)
    zeros128 = jnp.zeros((NPAD, HID), jnp.float32)

    hist = _sc_degree(dst2d, ones128, zeros128)
    xw, y = _tc_pre(x, W_gcn, hist)
    agg = _sc_aggregate(src2, dst2, y, zeros128)
    z_node, zg_sum, cnt = _tc_encode(
        agg, xw, hist,
        b_gcn.reshape(1, HID), W_enc, b_enc.reshape(1, LAT),
        batch.reshape(NBL, 1, RB),
    )
    a_hat, x_hat, z_graph = _tc_decode(
        z_node, zg_sum, cnt, batch.reshape(NBC, 1, RC),
        W_dec, b_dec.reshape(1, IN_CH),
    )
    return z_node, z_graph, x_hat, a_hat


# speedup vs baseline: 21.5646x; 1.1627x over previous
import functools

import jax
import jax.numpy as jnp
from jax import lax
from jax.experimental import pallas as pl
from jax.experimental.pallas import tpu as pltpu
from jax.experimental.pallas import tpu_sc as plsc

N = 10000
E = 320000
IN_CH = 128
HID = 128
LAT = 64
G = 64

NC = 2
NS = 16
NW = NC * NS
EPW = E // NW
CH = 40
NCHUNK = EPW // CH
NPAD = 10240
RPT = NPAD // NS
ROW0 = 624
RPW = 640

_sc_mesh = plsc.VectorSubcoreMesh(
    core_axis_name="c", subcore_axis_name="s", num_cores=NC, num_subcores=NS
)


@functools.partial(
    pl.kernel,
    out_type=jax.ShapeDtypeStruct((NC * N * HID,), jnp.float32),
    mesh=_sc_mesh,
    compiler_params=pltpu.CompilerParams(needs_layout_passes=False),
    scratch_types=[
        pltpu.VMEM((EPW,), jnp.int32),
        pltpu.VMEM((N,), jnp.float32),
        pltpu.VMEM((RPW,), jnp.float32),
        pltpu.VMEM((RPW,), jnp.float32),
        pltpu.VMEM((80 * HID,), jnp.float32),
        pltpu.VMEM_SHARED((NS * N,), jnp.float32),
    ],
)
def _sc_degree(dst_hbm, out_hbm, idx_v, hist_v, tbuf_v, acc_v, exp_v, hist_all):
    c = lax.axis_index("c")
    s = lax.axis_index("s")
    wid = s * NC + c
    z16 = jnp.zeros((16,), jnp.float32)

    def zero_hist(k, carry):
        hist_v[pl.ds(k * 16, 16)] = z16
        return carry

    lax.fori_loop(0, N // 16, zero_hist, 0)
    pltpu.sync_copy(dst_hbm.at[pl.ds(wid * EPW, EPW)], idx_v)

    def count(k, carry):
        idx16 = idx_v[pl.ds(k * 16, 16)]
        cnt, last = plsc.scan_count(idx16)
        plsc.addupdate_scatter(hist_v, [idx16], cnt.astype(jnp.float32), mask=last)
        return carry

    lax.fori_loop(0, EPW // 16, count, 0)

    pltpu.sync_copy(hist_v, hist_all.at[pl.ds(s * N, N)])
    plsc.subcore_barrier()

    off = s * ROW0

    def zero_acc(k, carry):
        acc_v[pl.ds(k * 16, 16)] = z16
        return carry

    lax.fori_loop(0, RPW // 16, zero_acc, 0)
    for w in range(NS):
        pltpu.sync_copy(hist_all.at[pl.ds(w * N + off, RPW)], tbuf_v)

        def addw(k, carry):
            acc_v[pl.ds(k * 16, 16)] += tbuf_v[pl.ds(k * 16, 16)]
            return carry

        lax.fori_loop(0, RPW // 16, addw, 0)

    for p in range(RPW // 80):
        def expand(m, carry):
            bvec = plsc.load_gather(
                acc_v, [jnp.full((16,), p * 80 + m, jnp.int32)]
            )
            for j in range(8):
                exp_v[pl.ds(m * HID + j * 16, 16)] = bvec
            return carry

        lax.fori_loop(0, 80, expand, 0)
        pltpu.sync_copy(
            exp_v,
            out_hbm.at[pl.ds(c * N * HID + (off + p * 80) * HID, 80 * HID)],
        )


@functools.partial(
    pl.kernel,
    out_type=jax.ShapeDtypeStruct((NC, N, HID), jnp.float32),
    mesh=_sc_mesh,
    scratch_types=[
        pltpu.VMEM((2, NCHUNK // 5, CH), jnp.int32),
        pltpu.VMEM((5, CH, HID), jnp.float32),
        [pltpu.SemaphoreType.DMA] * 5,
        [pltpu.SemaphoreType.DMA] * 5,
        pltpu.VMEM_SHARED((N, HID), jnp.float32),
    ],
)
def _sc_aggregate(
    src_hbm, dst_hbm, y_hbm, zeros_hbm, out_hbm,
    sd_v, rows5, gsem, ssem, agg_s,
):
    c = lax.axis_index("c")
    s = lax.axis_index("s")
    wid = s * NC + c
    NH = NCHUNK // 5
    NB = 5
    R0 = 624

    @pl.when(s < NS - 1)
    def _():
        pltpu.sync_copy(zeros_hbm.at[pl.ds(s * R0, R0)], agg_s.at[pl.ds(s * R0, R0)])

    @pl.when(s == NS - 1)
    def _():
        pltpu.sync_copy(
            zeros_hbm.at[pl.ds((NS - 1) * R0, N - (NS - 1) * R0)],
            agg_s.at[pl.ds((NS - 1) * R0, N - (NS - 1) * R0)],
        )

    plsc.subcore_barrier()

    for h in range(5):
        pltpu.sync_copy(src_hbm.at[wid, h], sd_v.at[0])
        pltpu.sync_copy(dst_hbm.at[wid, h], sd_v.at[1])

        for b in range(NB):
            pltpu.async_copy(y_hbm.at[sd_v.at[0, b]], rows5.at[b], gsem[b])

        def group(g, carry):
            base = g * NB
            for b in range(NB):
                ci = base + b
                pltpu.make_async_copy(
                    y_hbm.at[sd_v.at[0, ci]], rows5.at[b], gsem[b]
                ).wait()
                pltpu.async_copy(
                    rows5.at[b], agg_s.at[sd_v.at[1, ci]], ssem[b], add=True
                )
            for b in range(NB):
                ci = base + b
                pltpu.make_async_copy(
                    rows5.at[b], agg_s.at[sd_v.at[1, ci]], ssem[b]
                ).wait()

                @pl.when(base + NB + b < NH)
                def _():
                    pltpu.async_copy(
                        y_hbm.at[sd_v.at[0, base + NB + b]], rows5.at[b], gsem[b]
                    )
            return carry

        lax.fori_loop(0, NH // NB, group, 0)
    plsc.subcore_barrier()

    @pl.when(s < NS - 1)
    def _():
        pltpu.sync_copy(
            agg_s.at[pl.ds(s * R0, R0)], out_hbm.at[c, pl.ds(s * R0, R0)]
        )

    @pl.when(s == NS - 1)
    def _():
        pltpu.sync_copy(
            agg_s.at[pl.ds((NS - 1) * R0, N - (NS - 1) * R0)],
            out_hbm.at[c, pl.ds((NS - 1) * R0, N - (NS - 1) * R0)],
        )


RB = 1000
NBL = N // RB


def _tc_pre_body(x_ref, w_ref, h0_ref, h1_ref, xw_ref, y_ref):
    deg = h0_ref[0][:, :1] + h1_ref[0][:, :1] + 1.0
    xw = jnp.dot(x_ref[...], w_ref[...], preferred_element_type=jnp.float32)
    xw_ref[...] = xw
    y_ref[...] = xw * lax.rsqrt(deg)


def _tc_pre(x, W, hist):
    return pl.pallas_call(
        _tc_pre_body,
        grid=(NBL,),
        in_specs=[
            pl.BlockSpec((RB, IN_CH), lambda i: (i, 0)),
            pl.BlockSpec((IN_CH, HID), lambda i: (0, 0)),
            pl.BlockSpec((1, RB, HID), lambda i: (0, i, 0)),
            pl.BlockSpec((1, RB, HID), lambda i: (1, i, 0)),
        ],
        out_specs=[
            pl.BlockSpec((RB, HID), lambda i: (i, 0)),
            pl.BlockSpec((RB, HID), lambda i: (i, 0)),
        ],
        out_shape=[
            jax.ShapeDtypeStruct((N, HID), jnp.float32),
            jax.ShapeDtypeStruct((N, HID), jnp.float32),
        ],
    )(x, W, hist, hist)


def _tc_enc_body(a0, a1, xw_ref, h0, h1, bg, we, be, b_ref, z_ref, zg_ref, cnt_ref):
    i = pl.program_id(0)
    deg = h0[0][:, :1] + h1[0][:, :1] + 1.0
    dis = lax.rsqrt(deg)
    out = dis * (a0[0] + a1[0]) + xw_ref[...] * (1.0 / deg) + bg[...]
    hrelu = jnp.maximum(out, 0.0)
    z = jnp.dot(hrelu, we[...], preferred_element_type=jnp.float32) + be[...]
    z_ref[...] = z
    b2 = b_ref[0]
    S = (lax.broadcasted_iota(jnp.int32, (G, RB), 0) == b2).astype(jnp.float32)
    zg = jnp.dot(S, z, preferred_element_type=jnp.float32)
    csum = jnp.sum(S, axis=1, keepdims=True)

    @pl.when(i == 0)
    def _():
        zg_ref[...] = jnp.zeros_like(zg_ref)
        cnt_ref[...] = jnp.zeros_like(cnt_ref)

    zg_ref[...] += zg
    cnt_ref[...] += jnp.broadcast_to(csum, (G, G))


def _tc_encode(agg, xw, hist, bg, We, be, batch3):
    return pl.pallas_call(
        _tc_enc_body,
        grid=(NBL,),
        in_specs=[
            pl.BlockSpec((1, RB, HID), lambda i: (0, i, 0)),
            pl.BlockSpec((1, RB, HID), lambda i: (1, i, 0)),
            pl.BlockSpec((RB, HID), lambda i: (i, 0)),
            pl.BlockSpec((1, RB, HID), lambda i: (0, i, 0)),
            pl.BlockSpec((1, RB, HID), lambda i: (1, i, 0)),
            pl.BlockSpec((1, HID), lambda i: (0, 0)),
            pl.BlockSpec((HID, LAT), lambda i: (0, 0)),
            pl.BlockSpec((1, LAT), lambda i: (0, 0)),
            pl.BlockSpec((1, 1, RB), lambda i: (i, 0, 0)),
        ],
        out_specs=[
            pl.BlockSpec((RB, LAT), lambda i: (i, 0)),
            pl.BlockSpec((G, LAT), lambda i: (0, 0)),
            pl.BlockSpec((G, G), lambda i: (0, 0)),
        ],
        out_shape=[
            jax.ShapeDtypeStruct((N, LAT), jnp.float32),
            jax.ShapeDtypeStruct((G, LAT), jnp.float32),
            jax.ShapeDtypeStruct((G, G), jnp.float32),
        ],
    )(agg, agg, xw, hist, hist, bg, We, be, batch3)


RC = 400
NBC = N // RC


def _tc_dec_body(zb, zfull, zgs, cnt, b_ref, wd, bd, a_ref, x_ref, zgr_ref):
    zgraph = zgs[...] / jnp.maximum(cnt[...], 1.0)
    zgr_ref[...] = zgraph
    b2 = b_ref[0]
    St = (lax.broadcasted_iota(jnp.int32, (G, RC), 0) == b2).astype(jnp.float32)
    zrep = lax.dot_general(
        St, zgraph, (((0,), (0,)), ((), ())), preferred_element_type=jnp.float32
    )
    x_ref[...] = (
        jnp.dot(zrep, wd[...], preferred_element_type=jnp.float32) + bd[...]
    )
    logits = lax.dot_general(
        zb[...], zfull[...], (((1,), (1,)), ((), ())),
        preferred_element_type=jnp.float32,
    )
    a_ref[...] = 1.0 / (1.0 + jnp.exp(-logits))


def _tc_decode(z_node, zgs, cnt, batch3, Wd, bd):
    return pl.pallas_call(
        _tc_dec_body,
        grid=(NBC,),
        in_specs=[
            pl.BlockSpec((RC, LAT), lambda i: (i, 0)),
            pl.BlockSpec((N, LAT), lambda i: (0, 0)),
            pl.BlockSpec((G, LAT), lambda i: (0, 0)),
            pl.BlockSpec((G, G), lambda i: (0, 0)),
            pl.BlockSpec((1, 1, RC), lambda i: (i, 0, 0)),
            pl.BlockSpec((LAT, IN_CH), lambda i: (0, 0)),
            pl.BlockSpec((1, IN_CH), lambda i: (0, 0)),
        ],
        out_specs=[
            pl.BlockSpec((RC, N), lambda i: (i, 0)),
            pl.BlockSpec((RC, IN_CH), lambda i: (i, 0)),
            pl.BlockSpec((G, LAT), lambda i: (0, 0)),
        ],
        out_shape=[
            jax.ShapeDtypeStruct((N, N), jnp.float32),
            jax.ShapeDtypeStruct((N, IN_CH), jnp.float32),
            jax.ShapeDtypeStruct((G, LAT), jnp.float32),
        ],
    )(z_node, z_node, zgs, cnt, batch3, Wd, bd)


def kernel(x, edge_index, batch, W_gcn, b_gcn, W_enc, b_enc, W_dec, b_dec):
    src2 = edge_index[0].reshape(NW, 5, NCHUNK // 5, CH)
    dst2 = edge_index[1].reshape(NW, 5, NCHUNK // 5, CH)

    zeros128 = jnp.zeros((N, HID), jnp.float32)

    hist = _sc_degree(edge_index[1]).reshape(NC, N, HID)
    xw, y = _tc_pre(x, W_gcn, hist)
    agg = _sc_aggregate(src2, dst2, y, zeros128)
    z_node, zg_sum, cnt = _tc_encode(
        agg, xw, hist,
        b_gcn.reshape(1, HID), W_enc, b_enc.reshape(1, LAT),
        batch.reshape(NBL, 1, RB),
    )
    a_hat, x_hat, z_graph = _tc_decode(
        z_node, zg_sum, cnt, batch.reshape(NBC, 1, RC),
        W_dec, b_dec.reshape(1, IN_CH),
    )
    return z_node, z_graph, x_hat, a_hat


# speedup vs baseline: 22.7546x; 1.0552x over previous
import functools

import jax
import jax.numpy as jnp
from jax import lax
from jax.experimental import pallas as pl
from jax.experimental.pallas import tpu as pltpu
from jax.experimental.pallas import tpu_sc as plsc

N = 10000
E = 320000
IN_CH = 128
HID = 128
LAT = 64
G = 64

NC = 2
NS = 16
NW = NC * NS
EPW = E // NW
CH = 40
NCHUNK = EPW // CH
NPAD = 10240
RPT = NPAD // NS
ROW0 = 624
RPW = 640

_sc_mesh = plsc.VectorSubcoreMesh(
    core_axis_name="c", subcore_axis_name="s", num_cores=NC, num_subcores=NS
)


@functools.partial(
    pl.kernel,
    out_type=jax.ShapeDtypeStruct((NC * N * HID,), jnp.float32),
    mesh=_sc_mesh,
    compiler_params=pltpu.CompilerParams(needs_layout_passes=False),
    scratch_types=[
        pltpu.VMEM((EPW,), jnp.int32),
        pltpu.VMEM((N,), jnp.float32),
        pltpu.VMEM((RPW,), jnp.float32),
        pltpu.VMEM((RPW,), jnp.float32),
        pltpu.VMEM((80 * HID,), jnp.float32),
        pltpu.VMEM_SHARED((NS * N,), jnp.float32),
    ],
)
def _sc_degree(dst_hbm, out_hbm, idx_v, hist_v, tbuf_v, acc_v, exp_v, hist_all):
    c = lax.axis_index("c")
    s = lax.axis_index("s")
    wid = s * NC + c
    z16 = jnp.zeros((16,), jnp.float32)

    def zero_hist(k, carry):
        hist_v[pl.ds(k * 16, 16)] = z16
        return carry

    lax.fori_loop(0, N // 16, zero_hist, 0)
    pltpu.sync_copy(dst_hbm.at[pl.ds(wid * EPW, EPW)], idx_v)

    def count(k, carry):
        idx16 = idx_v[pl.ds(k * 16, 16)]
        cnt, last = plsc.scan_count(idx16)
        plsc.addupdate_scatter(hist_v, [idx16], cnt.astype(jnp.float32), mask=last)
        return carry

    lax.fori_loop(0, EPW // 16, count, 0)

    pltpu.sync_copy(hist_v, hist_all.at[pl.ds(s * N, N)])
    plsc.subcore_barrier()

    off = s * ROW0

    def zero_acc(k, carry):
        acc_v[pl.ds(k * 16, 16)] = z16
        return carry

    lax.fori_loop(0, RPW // 16, zero_acc, 0)
    for w in range(NS):
        pltpu.sync_copy(hist_all.at[pl.ds(w * N + off, RPW)], tbuf_v)

        def addw(k, carry):
            acc_v[pl.ds(k * 16, 16)] += tbuf_v[pl.ds(k * 16, 16)]
            return carry

        lax.fori_loop(0, RPW // 16, addw, 0)

    for p in range(RPW // 80):
        def expand(m, carry):
            bvec = plsc.load_gather(
                acc_v, [jnp.full((16,), p * 80 + m, jnp.int32)]
            )
            for j in range(8):
                exp_v[pl.ds(m * HID + j * 16, 16)] = bvec
            return carry

        lax.fori_loop(0, 80, expand, 0)
        pltpu.sync_copy(
            exp_v,
            out_hbm.at[pl.ds(c * N * HID + (off + p * 80) * HID, 80 * HID)],
        )


@functools.partial(
    pl.kernel,
    out_type=jax.ShapeDtypeStruct((NC, N, HID), jnp.float32),
    mesh=_sc_mesh,
    scratch_types=[
        pltpu.VMEM((2, NCHUNK // 5, CH), jnp.int32),
        pltpu.VMEM((5, CH, HID), jnp.float32),
        [pltpu.SemaphoreType.DMA] * 5,
        [pltpu.SemaphoreType.DMA] * 5,
        pltpu.VMEM_SHARED((N, HID), jnp.float32),
    ],
)
def _sc_aggregate(
    src_hbm, dst_hbm, y_hbm, zeros_hbm, out_hbm,
    sd_v, rows5, gsem, ssem, agg_s,
):
    c = lax.axis_index("c")
    s = lax.axis_index("s")
    wid = s * NC + c
    NH = NCHUNK // 5
    NB = 5
    R0 = 624

    @pl.when(s < NS - 1)
    def _():
        pltpu.sync_copy(zeros_hbm.at[pl.ds(s * R0, R0)], agg_s.at[pl.ds(s * R0, R0)])

    @pl.when(s == NS - 1)
    def _():
        pltpu.sync_copy(
            zeros_hbm.at[pl.ds((NS - 1) * R0, N - (NS - 1) * R0)],
            agg_s.at[pl.ds((NS - 1) * R0, N - (NS - 1) * R0)],
        )

    plsc.subcore_barrier()

    for h in range(5):
        pltpu.sync_copy(src_hbm.at[wid, h], sd_v.at[0])
        pltpu.sync_copy(dst_hbm.at[wid, h], sd_v.at[1])

        for b in range(NB):
            pltpu.async_copy(y_hbm.at[sd_v.at[0, b]], rows5.at[b], gsem[b])

        def group(g, carry):
            base = g * NB
            for b in range(NB):
                ci = base + b
                pltpu.make_async_copy(
                    y_hbm.at[sd_v.at[0, ci]], rows5.at[b], gsem[b]
                ).wait()
                pltpu.async_copy(
                    rows5.at[b], agg_s.at[sd_v.at[1, ci]], ssem[b], add=True
                )
            for b in range(NB):
                ci = base + b
                pltpu.make_async_copy(
                    rows5.at[b], agg_s.at[sd_v.at[1, ci]], ssem[b]
                ).wait()

                @pl.when(base + NB + b < NH)
                def _():
                    pltpu.async_copy(
                        y_hbm.at[sd_v.at[0, base + NB + b]], rows5.at[b], gsem[b]
                    )
            return carry

        lax.fori_loop(0, NH // NB, group, 0)
    plsc.subcore_barrier()

    @pl.when(s < NS - 1)
    def _():
        pltpu.sync_copy(
            agg_s.at[pl.ds(s * R0, R0)], out_hbm.at[c, pl.ds(s * R0, R0)]
        )

    @pl.when(s == NS - 1)
    def _():
        pltpu.sync_copy(
            agg_s.at[pl.ds((NS - 1) * R0, N - (NS - 1) * R0)],
            out_hbm.at[c, pl.ds((NS - 1) * R0, N - (NS - 1) * R0)],
        )


RB = 1000
NBL = N // RB


def _tc_pre_body(x_ref, w_ref, h0_ref, h1_ref, xw_ref, y_ref):
    deg = h0_ref[0][:, :1] + h1_ref[0][:, :1] + 1.0
    xw = jnp.dot(x_ref[...], w_ref[...], preferred_element_type=jnp.float32)
    xw_ref[...] = xw
    y_ref[...] = xw * lax.rsqrt(deg)


def _tc_pre(x, W, hist):
    return pl.pallas_call(
        _tc_pre_body,
        grid=(NBL,),
        in_specs=[
            pl.BlockSpec((RB, IN_CH), lambda i: (i, 0)),
            pl.BlockSpec((IN_CH, HID), lambda i: (0, 0)),
            pl.BlockSpec((1, RB, HID), lambda i: (0, i, 0)),
            pl.BlockSpec((1, RB, HID), lambda i: (1, i, 0)),
        ],
        out_specs=[
            pl.BlockSpec((RB, HID), lambda i: (i, 0)),
            pl.BlockSpec((RB, HID), lambda i: (i, 0)),
        ],
        out_shape=[
            jax.ShapeDtypeStruct((N, HID), jnp.float32),
            jax.ShapeDtypeStruct((N, HID), jnp.float32),
        ],
    )(x, W, hist, hist)


def _tc_enc_body(a0, a1, xw_ref, h0, h1, bg, we, be, b_ref, z_ref, zg_ref, cnt_ref):
    i = pl.program_id(0)
    deg = h0[0][:, :1] + h1[0][:, :1] + 1.0
    dis = lax.rsqrt(deg)
    out = dis * (a0[0] + a1[0]) + xw_ref[...] * (1.0 / deg) + bg[...]
    hrelu = jnp.maximum(out, 0.0)
    z = jnp.dot(hrelu, we[...], preferred_element_type=jnp.float32) + be[...]
    z_ref[...] = z
    b2 = b_ref[0]
    S = (lax.broadcasted_iota(jnp.int32, (G, RB), 0) == b2).astype(jnp.float32)
    zg = jnp.dot(S, z, preferred_element_type=jnp.float32)
    csum = jnp.sum(S, axis=1, keepdims=True)

    @pl.when(i == 0)
    def _():
        zg_ref[...] = jnp.zeros_like(zg_ref)
        cnt_ref[...] = jnp.zeros_like(cnt_ref)

    zg_ref[...] += zg
    cnt_ref[...] += jnp.broadcast_to(csum, (G, G))


def _tc_encode(agg, xw, hist, bg, We, be, batch3):
    return pl.pallas_call(
        _tc_enc_body,
        grid=(NBL,),
        in_specs=[
            pl.BlockSpec((1, RB, HID), lambda i: (0, i, 0)),
            pl.BlockSpec((1, RB, HID), lambda i: (1, i, 0)),
            pl.BlockSpec((RB, HID), lambda i: (i, 0)),
            pl.BlockSpec((1, RB, HID), lambda i: (0, i, 0)),
            pl.BlockSpec((1, RB, HID), lambda i: (1, i, 0)),
            pl.BlockSpec((1, HID), lambda i: (0, 0)),
            pl.BlockSpec((HID, LAT), lambda i: (0, 0)),
            pl.BlockSpec((1, LAT), lambda i: (0, 0)),
            pl.BlockSpec((1, 1, RB), lambda i: (i, 0, 0)),
        ],
        out_specs=[
            pl.BlockSpec((RB, LAT), lambda i: (i, 0)),
            pl.BlockSpec((G, LAT), lambda i: (0, 0)),
            pl.BlockSpec((G, G), lambda i: (0, 0)),
        ],
        out_shape=[
            jax.ShapeDtypeStruct((N, LAT), jnp.float32),
            jax.ShapeDtypeStruct((G, LAT), jnp.float32),
            jax.ShapeDtypeStruct((G, G), jnp.float32),
        ],
    )(agg, agg, xw, hist, hist, bg, We, be, batch3)


RC = 400
NBC = N // RC


def _tc_dec_body(zb, zfull, zgs, cnt, b_ref, wd, bd, a_ref, x_ref, zgr_ref):
    zgraph = zgs[...] / jnp.maximum(cnt[...], 1.0)
    zgr_ref[...] = zgraph
    b2 = b_ref[0]
    St = (lax.broadcasted_iota(jnp.int32, (G, RC), 0) == b2).astype(jnp.float32)
    zrep = lax.dot_general(
        St, zgraph, (((0,), (0,)), ((), ())), preferred_element_type=jnp.float32
    )
    x_ref[...] = (
        jnp.dot(zrep, wd[...], preferred_element_type=jnp.float32) + bd[...]
    )
    logits = lax.dot_general(
        zb[...], zfull[...], (((1,), (1,)), ((), ())),
        preferred_element_type=jnp.float32,
    )
    a_ref[...] = 0.5 * jnp.tanh(0.5 * logits) + 0.5


def _tc_decode(z_node, zgs, cnt, batch3, Wd, bd):
    return pl.pallas_call(
        _tc_dec_body,
        grid=(NBC,),
        in_specs=[
            pl.BlockSpec((RC, LAT), lambda i: (i, 0)),
            pl.BlockSpec((N, LAT), lambda i: (0, 0)),
            pl.BlockSpec((G, LAT), lambda i: (0, 0)),
            pl.BlockSpec((G, G), lambda i: (0, 0)),
            pl.BlockSpec((1, 1, RC), lambda i: (i, 0, 0)),
            pl.BlockSpec((LAT, IN_CH), lambda i: (0, 0)),
            pl.BlockSpec((1, IN_CH), lambda i: (0, 0)),
        ],
        out_specs=[
            pl.BlockSpec((RC, N), lambda i: (i, 0)),
            pl.BlockSpec((RC, IN_CH), lambda i: (i, 0)),
            pl.BlockSpec((G, LAT), lambda i: (0, 0)),
        ],
        out_shape=[
            jax.ShapeDtypeStruct((N, N), jnp.float32),
            jax.ShapeDtypeStruct((N, IN_CH), jnp.float32),
            jax.ShapeDtypeStruct((G, LAT), jnp.float32),
        ],
    )(z_node, z_node, zgs, cnt, batch3, Wd, bd)


def kernel(x, edge_index, batch, W_gcn, b_gcn, W_enc, b_enc, W_dec, b_dec):
    src2 = edge_index[0].reshape(NW, 5, NCHUNK // 5, CH)
    dst2 = edge_index[1].reshape(NW, 5, NCHUNK // 5, CH)

    zeros128 = jnp.zeros((N, HID), jnp.float32)

    hist = _sc_degree(edge_index[1]).reshape(NC, N, HID)
    xw, y = _tc_pre(x, W_gcn, hist)
    agg = _sc_aggregate(src2, dst2, y, zeros128)
    z_node, zg_sum, cnt = _tc_encode(
        agg, xw, hist,
        b_gcn.reshape(1, HID), W_enc, b_enc.reshape(1, LAT),
        batch.reshape(NBL, 1, RB),
    )
    a_hat, x_hat, z_graph = _tc_decode(
        z_node, zg_sum, cnt, batch.reshape(NBC, 1, RC),
        W_dec, b_dec.reshape(1, IN_CH),
    )
    return z_node, z_graph, x_hat, a_hat
